# Initial kernel scaffold; baseline (speedup 1.0000x reference)
import jax
import jax.numpy as jnp
from jax.experimental import pallas as pl


def kernel(x, hyperedge_index, timesteps, W_enc, Wt_enc, bt_enc, W_dec, Wt_dec, bt_dec, W1, b1, W2, b2):
    raise NotImplementedError("write your pallas kernel here")


# speedup vs baseline: 18.1387x; 18.1387x over previous
import functools
import math

import numpy as np
import jax
import jax.numpy as jnp
from jax import lax
from jax.experimental import pallas as pl
from jax.experimental.pallas import tpu as pltpu
from jax.experimental.pallas import tpu_sc as plsc

N0 = 100000
E0 = 1600000
INNER = 16
TEMB = 64

NUM_CORES = 2
NUM_SUBCORES = 16
NUM_WORKERS = NUM_CORES * NUM_SUBCORES
WIN = 128
WPC = 25
CHUNKS = 16
PER_TILE = WIN * WPC * CHUNKS
E_PAD = PER_TILE * NUM_WORKERS
R_DUMMY = 2016
NP = N0 + R_DUMMY
RPT = NP // NUM_SUBCORES
RL = NP * 4 // 128
IDX_ROWS = E_PAD // WIN


def _sc_propagate(table_aug, gidx2d, sidx2d, zeros_tab):
    mesh = plsc.VectorSubcoreMesh(core_axis_name="c", subcore_axis_name="s")

    @functools.partial(
        pl.kernel,
        out_type=jax.ShapeDtypeStruct((NUM_CORES, NUM_SUBCORES, RPT, 4),
                                      jnp.float32),
        mesh=mesh,
        compiler_params=pltpu.CompilerParams(use_tc_tiling_on_sc=False),
        scratch_types=[
            pltpu.VMEM((WIN,), jnp.int32),
            pltpu.VMEM((WIN,), jnp.int32),
            pltpu.VMEM((WIN, 4), jnp.float32),
            pltpu.VMEM((RPT, 4), jnp.float32),
            pltpu.VMEM_SHARED((NP, 4), jnp.float32),
            pltpu.VMEM_SHARED((NP, 4), jnp.float32),
        ],
    )
    def body(tab_hbm, gidx_hbm, sidx_hbm, zero_hbm, out_hbm,
             gwin, swin, rows, stage, table, acc):
        c = lax.axis_index("c")
        s = lax.axis_index("s")
        wid = s * NUM_CORES + c
        r0 = s * RPT
        pltpu.sync_copy(tab_hbm.at[s], stage)
        pltpu.sync_copy(stage, table.at[pl.ds(r0, RPT)])
        pltpu.sync_copy(zero_hbm.at[s], stage)
        pltpu.sync_copy(stage, acc.at[pl.ds(r0, RPT)])
        plsc.subcore_barrier()

        base = wid * PER_TILE

        def win_body(w, carry):
            off = base + w * WIN
            pltpu.sync_copy(gidx_hbm.at[pl.ds(off, WIN)], gwin)
            pltpu.sync_copy(sidx_hbm.at[pl.ds(off, WIN)], swin)
            pltpu.sync_copy(table.at[gwin], rows)
            pltpu.sync_copy(rows, acc.at[swin], add=True)
            return carry

        lax.fori_loop(0, PER_TILE // WIN, win_body, 0)
        plsc.subcore_barrier()
        pltpu.sync_copy(acc.at[pl.ds(r0, RPT)], stage)
        pltpu.sync_copy(stage, out_hbm.at[c, s])

    out = body(table_aug, gidx2d, sidx2d, zeros_tab)
    return out.reshape(NUM_CORES, NP, 4)


_GBLK = np.zeros((4, 4), np.float32)
_GBLK[3, :] = 1.0
_GNP = np.kron(np.eye(32, dtype=np.float32), _GBLK)


def _lane_masks():
    lane = lax.broadcasted_iota(jnp.int32, (1, 128), 1)
    is3 = (lane % 4) == 3
    return jnp.where(is3, 0.0, 1.0), jnp.where(is3, 1.0, 0.0)


def _combine_scale_body(p_ref, g_ref, o_ref):
    m012, m3 = _lane_masks()
    p = p_ref[0] + p_ref[1]
    spread = jnp.dot(p, g_ref[...], preferred_element_type=jnp.float32)
    scale = jnp.where(spread > 0, 1.0 / jnp.where(spread > 0, spread, 1.0), 0.0)
    o_ref[...] = p * scale * m012 + m3


def _combine_scale(partials):
    return pl.pallas_call(
        _combine_scale_body,
        out_shape=jax.ShapeDtypeStruct((RL, 128), jnp.float32),
    )(partials, jnp.asarray(_GNP))


def _dense_mid_body(p_ref, g_ref, a_ref, b_ref, tenc_ref, o_ref):
    m012, m3 = _lane_masks()
    p = p_ref[0] + p_ref[1]
    spread = jnp.dot(p, g_ref[...], preferred_element_type=jnp.float32)
    scale = jnp.where(spread > 0, 1.0 / jnp.where(spread > 0, spread, 1.0), 0.0)
    xs = p * scale * m012
    h = jax.nn.silu(jnp.dot(xs, a_ref[...], preferred_element_type=jnp.float32)
                    + tenc_ref[...])
    y = jnp.dot(h, b_ref[...], preferred_element_type=jnp.float32)
    o_ref[...] = y + m3


def _dense_mid(partials, a_bd, b_bd, tenc_tile):
    return pl.pallas_call(
        _dense_mid_body,
        out_shape=jax.ShapeDtypeStruct((RL, 128), jnp.float32),
    )(partials, jnp.asarray(_GNP), a_bd, b_bd, tenc_tile)


def _final_body(p_ref, g_ref, tdec_ref, o_ref):
    m012, _ = _lane_masks()
    p = p_ref[0] + p_ref[1]
    spread = jnp.dot(p, g_ref[...], preferred_element_type=jnp.float32)
    scale = jnp.where(spread > 0, 1.0 / jnp.where(spread > 0, spread, 1.0), 0.0)
    o_ref[...] = jax.nn.silu(p * scale * m012 + tdec_ref[...])


def _final(partials, tdec_tile):
    return pl.pallas_call(
        _final_body,
        out_shape=jax.ShapeDtypeStruct((RL, 128), jnp.float32),
    )(partials, jnp.asarray(_GNP), tdec_tile)


def _temb_body(ts_ref, w1_ref, b1_ref, w2_ref, b2_ref, wte_ref, bte_ref,
               wtd_ref, btd_ref, tenc_ref, tdec_ref):
    t = ts_ref[0].astype(jnp.float32)
    half = INNER // 2
    i8 = lax.broadcasted_iota(jnp.int32, (1, half), 1).astype(jnp.float32)
    freqs = jnp.exp(-math.log(10000.0) / half * i8)
    args = t * freqs
    temb = jnp.concatenate([jnp.cos(args), jnp.sin(args)], axis=1)
    emb = jnp.dot(jax.nn.silu(jnp.dot(temb, w1_ref[...],
                                      preferred_element_type=jnp.float32)
                              + b1_ref[...]),
                  w2_ref[...], preferred_element_type=jnp.float32) + b2_ref[...]
    se = jax.nn.silu(emb)
    tenc = jnp.dot(se, wte_ref[...], preferred_element_type=jnp.float32) + bte_ref[...]
    tdec = jnp.dot(se, wtd_ref[...], preferred_element_type=jnp.float32) + btd_ref[...]
    tenc_ref[...] = jnp.tile(tenc, (1, 32))
    tdec4 = jnp.concatenate([tdec, jnp.zeros((1, 1), jnp.float32)], axis=1)
    tdec_ref[...] = jnp.tile(tdec4, (1, 32))


def _temb(timesteps, W1, b1, W2, b2, Wt_enc, bt_enc, Wt_dec, bt_dec):
    return pl.pallas_call(
        _temb_body,
        out_shape=(jax.ShapeDtypeStruct((1, 512), jnp.float32),
                   jax.ShapeDtypeStruct((1, 128), jnp.float32)),
        in_specs=[pl.BlockSpec(memory_space=pltpu.SMEM)] + [pl.BlockSpec()] * 8,
    )(timesteps, W1, b1.reshape(1, TEMB), W2, b2.reshape(1, TEMB),
      Wt_enc, bt_enc.reshape(1, INNER), Wt_dec, bt_dec.reshape(1, 3))


def kernel(x, hyperedge_index, timesteps, W_enc, Wt_enc, bt_enc,
           W_dec, Wt_dec, bt_dec, W1, b1, W2, b2):
    node = hyperedge_index[0]
    edge = hyperedge_index[1]

    pad = E_PAD - E0
    dummy = (N0 + (jnp.arange(pad, dtype=jnp.int32) % R_DUMMY)).astype(jnp.int32)
    node_p = jnp.concatenate([node, dummy])
    edge_p = jnp.concatenate([edge, dummy])

    x_aug = jnp.concatenate([x, jnp.ones((N0, 1), x.dtype)], axis=1)
    x_aug = jnp.concatenate([x_aug, jnp.zeros((R_DUMMY, 4), x.dtype)], axis=0)
    x_aug = x_aug.reshape(NUM_SUBCORES, RPT, 4)
    zeros_tab = jnp.zeros((NUM_SUBCORES, RPT, 4), jnp.float32)

    eye32 = jnp.eye(32, dtype=jnp.float32)
    w_enc4 = jnp.concatenate([W_enc, jnp.zeros((1, INNER), jnp.float32)], axis=0)
    a_bd = jnp.kron(eye32, w_enc4)
    w_dec4 = jnp.concatenate([W_dec, jnp.zeros((INNER, 1), jnp.float32)], axis=1)
    b_bd = jnp.kron(eye32, w_dec4)

    tenc_tile, tdec_tile = _temb(timesteps, W1, b1, W2, b2,
                                 Wt_enc, bt_enc, Wt_dec, bt_dec)

    tab_shape = (NUM_SUBCORES, RPT, 4)
    pa = _sc_propagate(x_aug, node_p, edge_p, zeros_tab)
    ef1 = _combine_scale(pa.reshape(NUM_CORES, RL, 128))
    pb = _sc_propagate(ef1.reshape(tab_shape), edge_p, node_p, zeros_tab)
    y = _dense_mid(pb.reshape(NUM_CORES, RL, 128), a_bd, b_bd, tenc_tile)
    pc = _sc_propagate(y.reshape(tab_shape), node_p, edge_p, zeros_tab)
    ef2 = _combine_scale(pc.reshape(NUM_CORES, RL, 128))
    pd = _sc_propagate(ef2.reshape(tab_shape), edge_p, node_p, zeros_tab)
    o = _final(pd.reshape(NUM_CORES, RL, 128), tdec_tile)
    return o.reshape(NP, 4)[:N0, :3]
